# Initial kernel scaffold; baseline (speedup 1.0000x reference)
import jax
import jax.numpy as jnp
from jax.experimental import pallas as pl


def kernel(x, edge_index, W1, b1, W2, b2, Wo, bo):
    raise NotImplementedError("write your pallas kernel here")


# speedup vs baseline: 19.3141x; 19.3141x over previous
import functools

import jax
import jax.numpy as jnp
from jax import lax
from jax.experimental import pallas as pl
from jax.experimental.pallas import tpu as pltpu
from jax.experimental.pallas import tpu_sc as plsc

N = 10000
NP = 10240
E = 320000
D = 128
C = 64
NC = 2
NS = 16
NW = NC * NS
EPW = E // NW
CH = 128
NCHUNK = 80
EPC = NCHUNK * CH
RPT = NP // NS
ZB = 128

def _mesh():
    return plsc.VectorSubcoreMesh(core_axis_name="c", subcore_axis_name="s",
                                  num_cores=NC, num_subcores=NS)


@functools.cache
def _deg_kernel_fn():
    @functools.partial(
        pl.kernel,
        out_type=jax.ShapeDtypeStruct((NC, NP, D), jnp.float32),
        mesh=_mesh(),
        scratch_types=[
            pltpu.VMEM((NCHUNK, CH), jnp.int32),
            pltpu.VMEM((ZB, D), jnp.float32),
            pltpu.VMEM((ZB, D), jnp.float32),
            pltpu.MemorySpace.VMEM_SHARED((NP, D), jnp.float32),
        ],
    )
    def _deg_kernel(dst_hbm, ones_hbm, zeros_hbm, deg_out, idx_v, ones_v,
                    stage_v, deg_sh):
        c = lax.axis_index("c")
        s = lax.axis_index("s")
        wid = c * NS + s
        rstart = s * RPT
        pltpu.sync_copy(ones_hbm, ones_v)

        @pl.when(c == 0)
        def _():
            for k in range(RPT // ZB):
                pltpu.sync_copy(ones_v, deg_sh.at[pl.ds(rstart + k * ZB, ZB)])

        @pl.when(c != 0)
        def _():
            pltpu.sync_copy(zeros_hbm, stage_v)
            for k in range(RPT // ZB):
                pltpu.sync_copy(stage_v,
                                deg_sh.at[pl.ds(rstart + k * ZB, ZB)])

        pltpu.sync_copy(dst_hbm.at[wid], idx_v)
        plsc.subcore_barrier()

        def chunk(j, carry):
            pltpu.sync_copy(ones_v, deg_sh.at[idx_v.at[j]], add=True)
            return carry

        lax.fori_loop(0, NCHUNK, chunk, 0)
        plsc.subcore_barrier()
        for k in range(RPT // ZB):
            pltpu.sync_copy(deg_sh.at[pl.ds(rstart + k * ZB, ZB)], stage_v)
            pltpu.sync_copy(stage_v,
                            deg_out.at[c, pl.ds(rstart + k * ZB, ZB)])

    return _deg_kernel


@functools.cache
def _agg_kernel_fn():
    @functools.partial(
        pl.kernel,
        out_type=jax.ShapeDtypeStruct((NC, NP, D), jnp.float32),
        mesh=_mesh(),
        scratch_types=[
            pltpu.VMEM((NCHUNK, CH), jnp.int32),
            pltpu.VMEM((NCHUNK, CH), jnp.int32),
            pltpu.VMEM((CH, D), jnp.float32),
            pltpu.MemorySpace.VMEM_SHARED((NP, D), jnp.float32),
            pltpu.SemaphoreType.DMA,
        ],
    )
    def _agg_kernel(hp_hbm, src_hbm, dst_hbm, zrows_hbm, agg_out, sidx_v,
                    didx_v, rows_v, agg_sh, sem):
        c = lax.axis_index("c")
        s = lax.axis_index("s")
        wid = c * NS + s
        rstart = s * RPT

        @pl.when(c == 0)
        def _():
            for k in range(RPT // ZB):
                pltpu.sync_copy(hp_hbm.at[pl.ds(rstart + k * ZB, ZB)],
                                rows_v)
                pltpu.sync_copy(rows_v, agg_sh.at[pl.ds(rstart + k * ZB, ZB)])

        @pl.when(c != 0)
        def _():
            pltpu.sync_copy(zrows_hbm, rows_v)
            for k in range(RPT // ZB):
                pltpu.sync_copy(rows_v,
                                agg_sh.at[pl.ds(rstart + k * ZB, ZB)])

        pltpu.sync_copy(src_hbm.at[wid], sidx_v)
        pltpu.sync_copy(dst_hbm.at[wid], didx_v)
        plsc.subcore_barrier()

        def chunk(j, carry):
            pltpu.async_copy(hp_hbm.at[sidx_v.at[j]], rows_v, sem).wait()
            pltpu.sync_copy(rows_v, agg_sh.at[didx_v.at[j]], add=True)
            return carry

        lax.fori_loop(0, NCHUNK, chunk, 0)
        plsc.subcore_barrier()
        for k in range(RPT // ZB):
            pltpu.sync_copy(agg_sh.at[pl.ds(rstart + k * ZB, ZB)], rows_v)
            pltpu.sync_copy(rows_v,
                            agg_out.at[c, pl.ds(rstart + k * ZB, ZB)])

    return _agg_kernel


_BR = 1024


def _dinv_from(deg_ref):
    deg = deg_ref[0, :, 0] + deg_ref[1, :, 0]
    return lax.rsqrt(deg)


def _stage_a_body(x_ref, w_ref, deg_ref, out_ref):
    dinv = _dinv_from(deg_ref)
    h = jnp.dot(x_ref[...], w_ref[...], preferred_element_type=jnp.float32)
    out_ref[...] = h * dinv[:, None]


def _stage_mid_body(agg_ref, deg_ref, b_ref, w_ref, out_ref):
    dinv = _dinv_from(deg_ref)
    a = (agg_ref[0] + agg_ref[1]) * dinv[:, None] + b_ref[0]
    h = jnp.maximum(a, 0.0)
    hw = jnp.dot(h, w_ref[...], preferred_element_type=jnp.float32)
    out_ref[...] = hw * dinv[:, None]


def _stage_out_body(agg_ref, deg_ref, b_ref, wo_ref, bo_ref, out_ref):
    dinv = _dinv_from(deg_ref)
    a = (agg_ref[0] + agg_ref[1]) * dinv[:, None] + b_ref[0]
    h = jnp.maximum(a, 0.0)
    z = jnp.dot(h, wo_ref[...], preferred_element_type=jnp.float32) + bo_ref[0]
    m = jnp.max(z, axis=1, keepdims=True)
    zs = z - m
    out_ref[...] = zs - jnp.log(jnp.sum(jnp.exp(zs), axis=1, keepdims=True))


def _stage_a(x, w1, deg):
    grid = NP // _BR
    return pl.pallas_call(
        _stage_a_body,
        out_shape=jax.ShapeDtypeStruct((NP, D), jnp.float32),
        grid=(grid,),
        in_specs=[
            pl.BlockSpec((_BR, D), lambda i: (i, 0)),
            pl.BlockSpec((D, D), lambda i: (0, 0)),
            pl.BlockSpec((NC, _BR, D), lambda i: (0, i, 0)),
        ],
        out_specs=pl.BlockSpec((_BR, D), lambda i: (i, 0)),
    )(x, w1, deg)


def _stage_mid(agg, deg, b, w):
    grid = NP // _BR
    return pl.pallas_call(
        _stage_mid_body,
        out_shape=jax.ShapeDtypeStruct((NP, D), jnp.float32),
        grid=(grid,),
        in_specs=[
            pl.BlockSpec((NC, _BR, D), lambda i: (0, i, 0)),
            pl.BlockSpec((NC, _BR, D), lambda i: (0, i, 0)),
            pl.BlockSpec((1, D), lambda i: (0, 0)),
            pl.BlockSpec((D, D), lambda i: (0, 0)),
        ],
        out_specs=pl.BlockSpec((_BR, D), lambda i: (i, 0)),
    )(agg, deg, b, w)


def _stage_out(agg, deg, b, wo, bo):
    grid = NP // _BR
    return pl.pallas_call(
        _stage_out_body,
        out_shape=jax.ShapeDtypeStruct((NP, C), jnp.float32),
        grid=(grid,),
        in_specs=[
            pl.BlockSpec((NC, _BR, D), lambda i: (0, i, 0)),
            pl.BlockSpec((NC, _BR, D), lambda i: (0, i, 0)),
            pl.BlockSpec((1, D), lambda i: (0, 0)),
            pl.BlockSpec((D, C), lambda i: (0, 0)),
            pl.BlockSpec((1, C), lambda i: (0, 0)),
        ],
        out_specs=pl.BlockSpec((_BR, C), lambda i: (i, 0)),
    )(agg, deg, b, wo, bo)


def kernel(x, edge_index, W1, b1, W2, b2, Wo, bo):
    src = edge_index[0].astype(jnp.int32).reshape(NW, EPW)
    dst = edge_index[1].astype(jnp.int32).reshape(NW, EPW)
    padv = N + jnp.arange(EPC - EPW, dtype=jnp.int32) % (NP - N)
    padm = jnp.broadcast_to(padv, (NW, EPC - EPW))
    src = jnp.concatenate([src, padm], axis=1).reshape(NW, NCHUNK, CH)
    dst = jnp.concatenate([dst, padm], axis=1).reshape(NW, NCHUNK, CH)
    ones128 = jnp.ones((ZB, D), jnp.float32)
    zrows = jnp.zeros((ZB, D), jnp.float32)
    xp = jnp.pad(x, ((0, NP - N), (0, 0)))
    b1r = b1.reshape(1, D)
    b2r = b2.reshape(1, D)
    bor = bo.reshape(1, C)

    deg = _deg_kernel_fn()(dst, ones128, zrows)
    h1p = _stage_a(xp, W1, deg)
    agg1 = _agg_kernel_fn()(h1p, src, dst, zrows)
    h2p = _stage_mid(agg1, deg, b1r, W2)
    agg2 = _agg_kernel_fn()(h2p, src, dst, zrows)
    return _stage_out(agg2, deg, b2r, Wo, bor)[:N]


# speedup vs baseline: 26.0765x; 1.3501x over previous
import functools

import jax
import jax.numpy as jnp
from jax import lax
from jax.experimental import pallas as pl
from jax.experimental.pallas import tpu as pltpu
from jax.experimental.pallas import tpu_sc as plsc

N = 10000
NP = 10240
E = 320000
D = 128
C = 64
NC = 2
NS = 16
NW = NC * NS
EPW = E // NW
CH = 128
NCHUNK = 80
EPC = NCHUNK * CH
RPT = NP // NS
ZB = 128

def _mesh():
    return plsc.VectorSubcoreMesh(core_axis_name="c", subcore_axis_name="s",
                                  num_cores=NC, num_subcores=NS)


@functools.cache
def _deg_kernel_fn():
    @functools.partial(
        pl.kernel,
        out_type=jax.ShapeDtypeStruct((NC, NP, D), jnp.float32),
        mesh=_mesh(),
        scratch_types=[
            pltpu.VMEM((NCHUNK, CH), jnp.int32),
            pltpu.VMEM((ZB, D), jnp.float32),
            pltpu.VMEM((ZB, D), jnp.float32),
            pltpu.MemorySpace.VMEM_SHARED((NP, D), jnp.float32),
        ],
    )
    def _deg_kernel(dst_hbm, ones_hbm, zeros_hbm, deg_out, idx_v, ones_v,
                    stage_v, deg_sh):
        c = lax.axis_index("c")
        s = lax.axis_index("s")
        wid = c * NS + s
        rstart = s * RPT
        pltpu.sync_copy(ones_hbm, ones_v)

        @pl.when(c == 0)
        def _():
            for k in range(RPT // ZB):
                pltpu.sync_copy(ones_v, deg_sh.at[pl.ds(rstart + k * ZB, ZB)])

        @pl.when(c != 0)
        def _():
            pltpu.sync_copy(zeros_hbm, stage_v)
            for k in range(RPT // ZB):
                pltpu.sync_copy(stage_v,
                                deg_sh.at[pl.ds(rstart + k * ZB, ZB)])

        pltpu.sync_copy(dst_hbm.at[wid], idx_v)
        plsc.subcore_barrier()

        def chunk(j, carry):
            pltpu.sync_copy(ones_v, deg_sh.at[idx_v.at[j]], add=True)
            return carry

        lax.fori_loop(0, NCHUNK, chunk, 0)
        plsc.subcore_barrier()
        for k in range(RPT // ZB):
            pltpu.sync_copy(deg_sh.at[pl.ds(rstart + k * ZB, ZB)], stage_v)
            pltpu.sync_copy(stage_v,
                            deg_out.at[c, pl.ds(rstart + k * ZB, ZB)])

    return _deg_kernel


@functools.cache
def _agg_kernel_fn():
    @functools.partial(
        pl.kernel,
        out_type=jax.ShapeDtypeStruct((NC, NP, D), jnp.float32),
        mesh=_mesh(),
        scratch_types=[
            pltpu.VMEM((NCHUNK // 2, CH), jnp.int32),
            pltpu.VMEM((NCHUNK // 2, CH), jnp.int32),
            pltpu.VMEM((CH, D), jnp.float32),
            pltpu.VMEM((CH, D), jnp.float32),
            pltpu.MemorySpace.VMEM_SHARED((NP, D), jnp.float32),
            pltpu.SemaphoreType.DMA,
            pltpu.SemaphoreType.DMA,
        ],
    )
    def _agg_kernel(hp_hbm, src_hbm, dst_hbm, zrows_hbm, agg_out, sidx_v,
                    didx_v, rows_a, rows_b, agg_sh, sem_a, sem_b):
        c = lax.axis_index("c")
        s = lax.axis_index("s")
        wid = c * NS + s
        rstart = s * RPT

        @pl.when(c == 0)
        def _():
            for k in range(RPT // ZB):
                pltpu.sync_copy(hp_hbm.at[pl.ds(rstart + k * ZB, ZB)],
                                rows_a)
                pltpu.sync_copy(rows_a, agg_sh.at[pl.ds(rstart + k * ZB, ZB)])

        @pl.when(c != 0)
        def _():
            pltpu.sync_copy(zrows_hbm, rows_a)
            for k in range(RPT // ZB):
                pltpu.sync_copy(rows_a,
                                agg_sh.at[pl.ds(rstart + k * ZB, ZB)])

        plsc.subcore_barrier()

        def start_gather(j, buf, sem):
            pltpu.make_async_copy(hp_hbm.at[sidx_v.at[j]], buf, sem).start()

        def wait_gather(buf, sem):
            pltpu.make_async_copy(hp_hbm.at[sidx_v.at[0]], buf, sem).wait()

        half = NCHUNK // 2
        for q in range(2):
            pltpu.sync_copy(src_hbm.at[wid, pl.ds(q * half, half)], sidx_v)
            pltpu.sync_copy(dst_hbm.at[wid, pl.ds(q * half, half)], didx_v)
            start_gather(0, rows_a, sem_a)

            def pair(i, carry):
                j = 2 * i
                start_gather(j + 1, rows_b, sem_b)
                wait_gather(rows_a, sem_a)
                pltpu.sync_copy(rows_a, agg_sh.at[didx_v.at[j]], add=True)

                @pl.when(j + 2 < half)
                def _():
                    start_gather(j + 2, rows_a, sem_a)

                wait_gather(rows_b, sem_b)
                pltpu.sync_copy(rows_b, agg_sh.at[didx_v.at[j + 1]], add=True)
                return carry

            lax.fori_loop(0, half // 2, pair, 0)
        plsc.subcore_barrier()
        for k in range(RPT // ZB):
            pltpu.sync_copy(agg_sh.at[pl.ds(rstart + k * ZB, ZB)], rows_a)
            pltpu.sync_copy(rows_a,
                            agg_out.at[c, pl.ds(rstart + k * ZB, ZB)])

    return _agg_kernel


_BR = 1024


def _dinv_from(deg_ref):
    deg = deg_ref[0, :, 0] + deg_ref[1, :, 0]
    return lax.rsqrt(deg)


def _stage_a_body(x_ref, w_ref, deg_ref, out_ref):
    dinv = _dinv_from(deg_ref)
    h = jnp.dot(x_ref[...], w_ref[...], preferred_element_type=jnp.float32)
    out_ref[...] = h * dinv[:, None]


def _stage_mid_body(agg_ref, deg_ref, b_ref, w_ref, out_ref):
    dinv = _dinv_from(deg_ref)
    a = (agg_ref[0] + agg_ref[1]) * dinv[:, None] + b_ref[0]
    h = jnp.maximum(a, 0.0)
    hw = jnp.dot(h, w_ref[...], preferred_element_type=jnp.float32)
    out_ref[...] = hw * dinv[:, None]


def _stage_out_body(agg_ref, deg_ref, b_ref, wo_ref, bo_ref, out_ref):
    dinv = _dinv_from(deg_ref)
    a = (agg_ref[0] + agg_ref[1]) * dinv[:, None] + b_ref[0]
    h = jnp.maximum(a, 0.0)
    z = jnp.dot(h, wo_ref[...], preferred_element_type=jnp.float32) + bo_ref[0]
    m = jnp.max(z, axis=1, keepdims=True)
    zs = z - m
    out_ref[...] = zs - jnp.log(jnp.sum(jnp.exp(zs), axis=1, keepdims=True))


def _stage_a(x, w1, deg):
    grid = NP // _BR
    return pl.pallas_call(
        _stage_a_body,
        out_shape=jax.ShapeDtypeStruct((NP, D), jnp.float32),
        grid=(grid,),
        in_specs=[
            pl.BlockSpec((_BR, D), lambda i: (i, 0)),
            pl.BlockSpec((D, D), lambda i: (0, 0)),
            pl.BlockSpec((NC, _BR, D), lambda i: (0, i, 0)),
        ],
        out_specs=pl.BlockSpec((_BR, D), lambda i: (i, 0)),
    )(x, w1, deg)


def _stage_mid(agg, deg, b, w):
    grid = NP // _BR
    return pl.pallas_call(
        _stage_mid_body,
        out_shape=jax.ShapeDtypeStruct((NP, D), jnp.float32),
        grid=(grid,),
        in_specs=[
            pl.BlockSpec((NC, _BR, D), lambda i: (0, i, 0)),
            pl.BlockSpec((NC, _BR, D), lambda i: (0, i, 0)),
            pl.BlockSpec((1, D), lambda i: (0, 0)),
            pl.BlockSpec((D, D), lambda i: (0, 0)),
        ],
        out_specs=pl.BlockSpec((_BR, D), lambda i: (i, 0)),
    )(agg, deg, b, w)


def _stage_out(agg, deg, b, wo, bo):
    grid = NP // _BR
    return pl.pallas_call(
        _stage_out_body,
        out_shape=jax.ShapeDtypeStruct((NP, C), jnp.float32),
        grid=(grid,),
        in_specs=[
            pl.BlockSpec((NC, _BR, D), lambda i: (0, i, 0)),
            pl.BlockSpec((NC, _BR, D), lambda i: (0, i, 0)),
            pl.BlockSpec((1, D), lambda i: (0, 0)),
            pl.BlockSpec((D, C), lambda i: (0, 0)),
            pl.BlockSpec((1, C), lambda i: (0, 0)),
        ],
        out_specs=pl.BlockSpec((_BR, C), lambda i: (i, 0)),
    )(agg, deg, b, wo, bo)


def kernel(x, edge_index, W1, b1, W2, b2, Wo, bo):
    src = edge_index[0].astype(jnp.int32).reshape(NW, EPW)
    dst = edge_index[1].astype(jnp.int32).reshape(NW, EPW)
    padv = N + jnp.arange(EPC - EPW, dtype=jnp.int32) % (NP - N)
    padm = jnp.broadcast_to(padv, (NW, EPC - EPW))
    src = jnp.concatenate([src, padm], axis=1).reshape(NW, NCHUNK, CH)
    dst = jnp.concatenate([dst, padm], axis=1).reshape(NW, NCHUNK, CH)
    ones128 = jnp.ones((ZB, D), jnp.float32)
    zrows = jnp.zeros((ZB, D), jnp.float32)
    xp = jnp.pad(x, ((0, NP - N), (0, 0)))
    b1r = b1.reshape(1, D)
    b2r = b2.reshape(1, D)
    bor = bo.reshape(1, C)

    deg = _deg_kernel_fn()(dst, ones128, zrows)
    h1p = _stage_a(xp, W1, deg)
    agg1 = _agg_kernel_fn()(h1p, src, dst, zrows)
    h2p = _stage_mid(agg1, deg, b1r, W2)
    agg2 = _agg_kernel_fn()(h2p, src, dst, zrows)
    return _stage_out(agg2, deg, b2r, Wo, bor)[:N]
